# Initial kernel scaffold; baseline (speedup 1.0000x reference)
import jax
import jax.numpy as jnp
from jax.experimental import pallas as pl


def kernel(x, edge_index, params):
    raise NotImplementedError("write your pallas kernel here")


# speedup vs baseline: 7.4467x; 7.4467x over previous
import functools

import jax
import jax.numpy as jnp
from jax import lax
from jax.experimental import pallas as pl
from jax.experimental.pallas import tpu as pltpu
from jax.experimental.pallas import tpu_sc as plsc

N = 10000
NP = 10240
E = 320000
EP = 323584
CH = 128
AW = 16
NSUB = 16
NCORE = 2
D_IN = 128
HD = 64
HEADS = 4
D_OUT = 6
RB = 128
GRID = NP // RB
ROWS_PER_TILE = NP // NSUB

_mesh = plsc.VectorSubcoreMesh(
    core_axis_name="c", subcore_axis_name="s",
    num_cores=NCORE, num_subcores=NSUB)

_sc_params = pltpu.CompilerParams(
    needs_layout_passes=False, use_tc_tiling_on_sc=False)


def _full_spec(shape):
    nd = len(shape)
    return pl.BlockSpec(shape, lambda i, _nd=nd: (0,) * _nd)


def _gelu(x):
    return 0.5 * x * (1.0 + lax.erf(x * 0.7071067811865476))


def _layer_norm(x, gamma, beta, eps=1e-5):
    mu = jnp.mean(x, axis=-1, keepdims=True)
    var = jnp.mean((x - mu) ** 2, axis=-1, keepdims=True)
    return (x - mu) / jnp.sqrt(var + eps) * gamma + beta


def _lrelu(x):
    return jnp.where(x >= 0, x, 0.2 * x)


def _tc_a_body(x_ref, pw, pb, h0w, h0b, h1w, h1b, h2w, h2b, qw, qb,
               lng, lnb, w0, aws, awd, xl0_ref, asrc_ref, adst_ref):
    x = x_ref[...]
    t = _gelu(jnp.dot(x, pw[...], preferred_element_type=jnp.float32) + pb[...])
    for (w, b) in ((h0w, h0b), (h1w, h1b), (h2w, h2b)):
        t = _gelu(jnp.dot(t, w[...], preferred_element_type=jnp.float32) + b[...]) + t
    h = jnp.dot(t, qw[...], preferred_element_type=jnp.float32) + qb[...]
    h = _layer_norm(h, lng[...], lnb[...])
    xl = jnp.dot(h, w0[...], preferred_element_type=jnp.float32)
    xl0_ref[0] = xl[:, :128]
    xl0_ref[1] = xl[:, 128:]
    asrc = jnp.dot(xl, aws[...], preferred_element_type=jnp.float32)
    adst = jnp.dot(xl, awd[...], preferred_element_type=jnp.float32)
    z = jnp.zeros((RB, AW - 2), jnp.float32)
    asrc_ref[0] = jnp.concatenate([asrc[:, 0:2], z], axis=1)
    asrc_ref[1] = jnp.concatenate([asrc[:, 2:4], z], axis=1)
    adst_ref[0] = jnp.concatenate([adst[:, 0:2], z], axis=1)
    adst_ref[1] = jnp.concatenate([adst[:, 2:4], z], axis=1)


def _tc_a(x_pad, enc, ln1, w0, aws, awd):
    weights = [enc['pre'][0], enc['pre'][1]]
    for (w, b) in enc['hidden']:
        weights += [w, b]
    weights += [enc['post'][0], enc['post'][1], ln1[0], ln1[1], w0, aws, awd]
    in_specs = [pl.BlockSpec((RB, D_IN), lambda i: (i, 0))]
    in_specs += [_full_spec(w.shape) for w in weights]
    return pl.pallas_call(
        _tc_a_body,
        grid=(GRID,),
        in_specs=in_specs,
        out_specs=[
            pl.BlockSpec((2, RB, 128), lambda i: (0, i, 0)),
            pl.BlockSpec((2, RB, AW), lambda i: (0, i, 0)),
            pl.BlockSpec((2, RB, AW), lambda i: (0, i, 0)),
        ],
        out_shape=[
            jax.ShapeDtypeStruct((2, NP, 128), jnp.float32),
            jax.ShapeDtypeStruct((2, NP, AW), jnp.float32),
            jax.ShapeDtypeStruct((2, NP, AW), jnp.float32),
        ],
    )(x_pad, *weights)


def _zero_vmem(ref, nwords):
    ncol = ref.shape[-1]
    iota = lax.broadcasted_iota(jnp.int32, (16,), 0)
    zero = jnp.zeros((16,), jnp.float32)

    @pl.loop(0, nwords // 16)
    def _(i):
        base = i * 16
        r = jnp.full((16,), base // ncol, jnp.int32)
        cvec = (base % ncol) + iota
        plsc.store_scatter(ref, [r, cvec], zero)


def _sc_conv0_body(xl0, asrc_h, adst_h, srcp, dstp, msg_out, den_out,
                   out_sp, den_sp, sidx, didx, didxa, msgb, exb, asb, adb,
                   gsem, asem):
    c = lax.axis_index("c")
    s = lax.axis_index("s")
    iota = lax.broadcasted_iota(jnp.int32, (16,), 0)
    z16 = jnp.zeros((16,), jnp.int32)
    o16 = jnp.ones((16,), jnp.int32)
    cnp = jnp.full((16,), c * NP, jnp.int32)

    _zero_vmem(msgb, CH * 128)
    _zero_vmem(exb, CH * AW)

    @pl.loop(0, ROWS_PER_TILE // CH)
    def _(k):
        rows = pl.ds(s * ROWS_PER_TILE + k * CH, CH)
        pltpu.sync_copy(msgb, out_sp.at[rows])
        pltpu.sync_copy(exb, den_sp.at[rows])

    plsc.subcore_barrier()

    ebase = s * (EP // NSUB)
    nchunks = EP // NSUB // CH

    @pl.loop(0, nchunks)
    def _(k):
        eb = ebase + k * CH
        pltpu.sync_copy(srcp.at[pl.ds(eb, CH)], sidx)
        pltpu.sync_copy(dstp.at[pl.ds(eb, CH)], didx)
        for g in range(8):
            sl = pl.ds(g * 16, 16)
            sidx[sl] = sidx[sl] + cnp
            didxa[sl] = didx[sl] + cnp
        gd = pltpu.async_copy(xl0.at[sidx], msgb, gsem)
        ga = pltpu.async_copy(asrc_h.at[sidx], asb, asem)
        gb = pltpu.async_copy(adst_h.at[didxa], adb, asem)
        ga.wait()
        gb.wait()

        ex0s, ex1s, evecs = [], [], []
        for g in range(8):
            rvec = g * 16 + iota
            a0 = (plsc.load_gather(asb, [rvec, z16])
                  + plsc.load_gather(adb, [rvec, z16]))
            a1 = (plsc.load_gather(asb, [rvec, o16])
                  + plsc.load_gather(adb, [rvec, o16]))
            ex0 = jnp.exp(_lrelu(a0))
            ex1 = jnp.exp(_lrelu(a1))
            plsc.store_scatter(exb, [rvec, z16], ex0)
            plsc.store_scatter(exb, [rvec, o16], ex1)
            ex0s.append(ex0)
            ex1s.append(ex1)
            evecs.append(rvec)
        gd.wait()

        @pl.loop(0, 64)
        def _(j):
            jf = jnp.full((16,), j, jnp.int32)
            for g in range(8):
                v = plsc.load_gather(msgb, [evecs[g], jf])
                plsc.store_scatter(msgb, [evecs[g], jf], v * ex0s[g])

        @pl.loop(64, 128)
        def _(j):
            jf = jnp.full((16,), j, jnp.int32)
            for g in range(8):
                v = plsc.load_gather(msgb, [evecs[g], jf])
                plsc.store_scatter(msgb, [evecs[g], jf], v * ex1s[g])

        pltpu.sync_copy(exb, den_sp.at[didx], add=True)
        pltpu.sync_copy(msgb, out_sp.at[didx], add=True)

    plsc.subcore_barrier()

    @pl.loop(0, ROWS_PER_TILE // CH)
    def _(k):
        rloc = pl.ds(s * ROWS_PER_TILE + k * CH, CH)
        rout = pl.ds(c * NP + s * ROWS_PER_TILE + k * CH, CH)
        pltpu.sync_copy(out_sp.at[rloc], msgb)
        pltpu.sync_copy(msgb, msg_out.at[rout])
        pltpu.sync_copy(den_sp.at[rloc], exb)
        pltpu.sync_copy(exb, den_out.at[rout])


_sc_conv0 = functools.partial(
    pl.kernel,
    out_type=[
        jax.ShapeDtypeStruct((2 * NP, 128), jnp.float32),
        jax.ShapeDtypeStruct((2 * NP, AW), jnp.float32),
    ],
    mesh=_mesh,
    scratch_types=[
        pltpu.VMEM_SHARED((NP, 128), jnp.float32),
        pltpu.VMEM_SHARED((NP, AW), jnp.float32),
        pltpu.VMEM((CH,), jnp.int32),
        pltpu.VMEM((CH,), jnp.int32),
        pltpu.VMEM((CH,), jnp.int32),
        pltpu.VMEM((CH, 128), jnp.float32),
        pltpu.VMEM((CH, AW), jnp.float32),
        pltpu.VMEM((CH, AW), jnp.float32),
        pltpu.VMEM((CH, AW), jnp.float32),
        pltpu.SemaphoreType.DMA,
        pltpu.SemaphoreType.DMA,
    ],
    compiler_params=_sc_params,
)(_sc_conv0_body)


def _sc_conv1_body(xl1, a1, srcp, dstp, msg_out, den_out,
                   out_sp, den_sp, sidx, didx, msgb, exb, asb, adb, gsem, asem):
    c = lax.axis_index("c")
    s = lax.axis_index("s")
    iota = lax.broadcasted_iota(jnp.int32, (16,), 0)
    z16 = jnp.zeros((16,), jnp.int32)
    o16 = jnp.ones((16,), jnp.int32)

    _zero_vmem(msgb, CH * HD)
    _zero_vmem(exb, CH * AW)

    @pl.loop(0, ROWS_PER_TILE // CH)
    def _(k):
        rows = pl.ds(s * ROWS_PER_TILE + k * CH, CH)
        pltpu.sync_copy(msgb, out_sp.at[rows])
        pltpu.sync_copy(exb, den_sp.at[rows])

    plsc.subcore_barrier()

    ebase = c * (EP // NCORE) + s * (EP // NCORE // NSUB)
    nchunks = EP // NCORE // NSUB // CH

    @pl.loop(0, nchunks)
    def _(k):
        eb = ebase + k * CH
        pltpu.sync_copy(srcp.at[pl.ds(eb, CH)], sidx)
        pltpu.sync_copy(dstp.at[pl.ds(eb, CH)], didx)
        gd = pltpu.async_copy(xl1.at[sidx], msgb, gsem)
        ga = pltpu.async_copy(a1.at[sidx], asb, asem)
        gb = pltpu.async_copy(a1.at[didx], adb, asem)
        ga.wait()
        gb.wait()

        exs, evecs = [], []
        for g in range(8):
            rvec = g * 16 + iota
            a = (plsc.load_gather(asb, [rvec, z16])
                 + plsc.load_gather(adb, [rvec, o16]))
            ex = jnp.exp(_lrelu(a))
            plsc.store_scatter(exb, [rvec, z16], ex)
            exs.append(ex)
            evecs.append(rvec)
        gd.wait()

        @pl.loop(0, HD)
        def _(j):
            jf = jnp.full((16,), j, jnp.int32)
            for g in range(8):
                v = plsc.load_gather(msgb, [evecs[g], jf])
                plsc.store_scatter(msgb, [evecs[g], jf], v * exs[g])

        pltpu.sync_copy(exb, den_sp.at[didx], add=True)
        pltpu.sync_copy(msgb, out_sp.at[didx], add=True)

    plsc.subcore_barrier()

    @pl.loop(0, ROWS_PER_TILE // CH)
    def _(k):
        rloc = pl.ds(s * ROWS_PER_TILE + k * CH, CH)
        rout = pl.ds(c * NP + s * ROWS_PER_TILE + k * CH, CH)
        pltpu.sync_copy(out_sp.at[rloc], msgb)
        pltpu.sync_copy(msgb, msg_out.at[rout])
        pltpu.sync_copy(den_sp.at[rloc], exb)
        pltpu.sync_copy(exb, den_out.at[rout])


_sc_conv1 = functools.partial(
    pl.kernel,
    out_type=[
        jax.ShapeDtypeStruct((2 * NP, HD), jnp.float32),
        jax.ShapeDtypeStruct((2 * NP, AW), jnp.float32),
    ],
    mesh=_mesh,
    scratch_types=[
        pltpu.VMEM_SHARED((NP, HD), jnp.float32),
        pltpu.VMEM_SHARED((NP, AW), jnp.float32),
        pltpu.VMEM((CH,), jnp.int32),
        pltpu.VMEM((CH,), jnp.int32),
        pltpu.VMEM((CH, HD), jnp.float32),
        pltpu.VMEM((CH, AW), jnp.float32),
        pltpu.VMEM((CH, AW), jnp.float32),
        pltpu.VMEM((CH, AW), jnp.float32),
        pltpu.SemaphoreType.DMA,
        pltpu.SemaphoreType.DMA,
    ],
    compiler_params=_sc_params,
)(_sc_conv1_body)


def _tc_b_body(msg_ref, den_ref, asrc_ref, adst_ref, xl0_ref,
               b0, w1, a1s, a1d, xl1_ref, a1_ref):
    outs = []
    for c in range(2):
        exs = jnp.exp(_lrelu(asrc_ref[c][:, 0:2] + adst_ref[c][:, 0:2]))
        den = den_ref[c][:, 0:2] + exs
        e2 = jnp.broadcast_to(exs[:, :, None], (RB, 2, HD)).reshape(RB, 128)
        d2 = jnp.broadcast_to(den[:, :, None], (RB, 2, HD)).reshape(RB, 128)
        outs.append((msg_ref[c] + e2 * xl0_ref[c]) / (d2 + 1e-16))
    out0 = jnp.concatenate(outs, axis=1) + b0[...]
    h = jnp.where(out0 > 0, out0, jnp.exp(jnp.minimum(out0, 0.0)) - 1.0)
    xl1 = jnp.dot(h, w1[...], preferred_element_type=jnp.float32)
    xl1_ref[...] = xl1
    a1_ref[...] = jnp.concatenate(
        [jnp.dot(xl1, a1s[...], preferred_element_type=jnp.float32),
         jnp.dot(xl1, a1d[...], preferred_element_type=jnp.float32),
         jnp.zeros((RB, AW - 2), jnp.float32)], axis=1)


def _tc_b(msg0, den0, asrc0, adst0, xl0, b0, w1, a1s, a1d):
    weights = [b0, w1, a1s, a1d]
    in_specs = [
        pl.BlockSpec((2, RB, 128), lambda i: (0, i, 0)),
        pl.BlockSpec((2, RB, AW), lambda i: (0, i, 0)),
        pl.BlockSpec((2, RB, AW), lambda i: (0, i, 0)),
        pl.BlockSpec((2, RB, AW), lambda i: (0, i, 0)),
        pl.BlockSpec((2, RB, 128), lambda i: (0, i, 0)),
    ] + [_full_spec(w.shape) for w in weights]
    return pl.pallas_call(
        _tc_b_body,
        grid=(GRID,),
        in_specs=in_specs,
        out_specs=[
            pl.BlockSpec((RB, HD), lambda i: (i, 0)),
            pl.BlockSpec((RB, AW), lambda i: (i, 0)),
        ],
        out_shape=[
            jax.ShapeDtypeStruct((NP, HD), jnp.float32),
            jax.ShapeDtypeStruct((NP, AW), jnp.float32),
        ],
    )(msg0, den0, asrc0, adst0, xl0, *weights)


def _tc_c_body(msg_ref, den_ref, xl1_ref, a1_ref, b1, lng, lnb,
               pw, pb, h0w, h0b, h1w, h1b, h2w, h2b, qw, qb, out_ref):
    exs = jnp.exp(_lrelu(a1_ref[:, 0:1] + a1_ref[:, 1:2]))
    den = den_ref[0][:, 0:1] + den_ref[1][:, 0:1] + exs
    out1 = (msg_ref[0] + msg_ref[1] + exs * xl1_ref[...]) / (den + 1e-16)
    out1 = out1 + b1[...]
    h = _layer_norm(out1, lng[...], lnb[...])
    t = _gelu(jnp.dot(h, pw[...], preferred_element_type=jnp.float32) + pb[...])
    for (w, b) in ((h0w, h0b), (h1w, h1b), (h2w, h2b)):
        t = _gelu(jnp.dot(t, w[...], preferred_element_type=jnp.float32) + b[...]) + t
    out_ref[...] = jnp.dot(t, qw[...], preferred_element_type=jnp.float32) + qb[...]


def _tc_c(msg1, den1, xl1, a1, b1, ln2, dec):
    weights = [b1, ln2[0], ln2[1], dec['pre'][0], dec['pre'][1]]
    for (w, b) in dec['hidden']:
        weights += [w, b]
    weights += [dec['post'][0], dec['post'][1]]
    in_specs = [
        pl.BlockSpec((2, RB, HD), lambda i: (0, i, 0)),
        pl.BlockSpec((2, RB, AW), lambda i: (0, i, 0)),
        pl.BlockSpec((RB, HD), lambda i: (i, 0)),
        pl.BlockSpec((RB, AW), lambda i: (i, 0)),
    ] + [_full_spec(w.shape) for w in weights]
    return pl.pallas_call(
        _tc_c_body,
        grid=(GRID,),
        in_specs=in_specs,
        out_specs=pl.BlockSpec((RB, D_OUT), lambda i: (i, 0)),
        out_shape=jax.ShapeDtypeStruct((NP, D_OUT), jnp.float32),
    )(msg1, den1, xl1, a1, *weights)


def kernel(x, edge_index, params):
    x_pad = jnp.pad(x, ((0, NP - N), (0, 0)))
    pad = jnp.full((EP - E,), N, jnp.int32)
    srcp = jnp.concatenate([edge_index[0], pad])
    dstp = jnp.concatenate([edge_index[1], pad])

    g0 = params['gat'][0]
    att_s = jnp.zeros((HEADS, HD, HEADS), jnp.float32)
    att_s = att_s.at[jnp.arange(HEADS), :, jnp.arange(HEADS)].set(g0['att_src'][0])
    att_d = jnp.zeros((HEADS, HD, HEADS), jnp.float32)
    att_d = att_d.at[jnp.arange(HEADS), :, jnp.arange(HEADS)].set(g0['att_dst'][0])
    aws = att_s.reshape(HEADS * HD, HEADS)
    awd = att_d.reshape(HEADS * HD, HEADS)

    xl0, asrc0, adst0 = _tc_a(x_pad, params['enc'], params['ln1'],
                              g0['W'], aws, awd)
    msg0, den0 = _sc_conv0(xl0.reshape(2 * NP, 128),
                           asrc0.reshape(2 * NP, AW),
                           adst0.reshape(2 * NP, AW), srcp, dstp)
    msg0 = msg0.reshape(2, NP, 128)
    den0 = den0.reshape(2, NP, AW)

    g1 = params['gat'][1]
    a1s = g1['att_src'].reshape(HD, 1)
    a1d = g1['att_dst'].reshape(HD, 1)
    xl1, a1 = _tc_b(msg0, den0, asrc0, adst0, xl0, g0['bias'],
                    g1['W'], a1s, a1d)
    msg1, den1 = _sc_conv1(xl1, a1, srcp, dstp)
    msg1 = msg1.reshape(2, NP, HD)
    den1 = den1.reshape(2, NP, AW)

    outp = _tc_c(msg1, den1, xl1, a1, g1['bias'], params['ln2'], params['dec'])
    return outp[:N]


# speedup vs baseline: 7.5257x; 1.0106x over previous
import functools

import jax
import jax.numpy as jnp
from jax import lax
from jax.experimental import pallas as pl
from jax.experimental.pallas import tpu as pltpu
from jax.experimental.pallas import tpu_sc as plsc

N = 10000
NP = 10240
E = 320000
EP = 323584
CH = 128
AW = 16
NSUB = 16
NCORE = 2
D_IN = 128
HD = 64
HEADS = 4
D_OUT = 6
RB = 128
GRID = NP // RB
ROWS_PER_TILE = NP // NSUB

_mesh = plsc.VectorSubcoreMesh(
    core_axis_name="c", subcore_axis_name="s",
    num_cores=NCORE, num_subcores=NSUB)

_sc_params = pltpu.CompilerParams(
    needs_layout_passes=False, use_tc_tiling_on_sc=False)


def _full_spec(shape):
    nd = len(shape)
    return pl.BlockSpec(shape, lambda i, _nd=nd: (0,) * _nd)


def _gelu(x):
    return 0.5 * x * (1.0 + lax.erf(x * 0.7071067811865476))


def _layer_norm(x, gamma, beta, eps=1e-5):
    mu = jnp.mean(x, axis=-1, keepdims=True)
    var = jnp.mean((x - mu) ** 2, axis=-1, keepdims=True)
    return (x - mu) / jnp.sqrt(var + eps) * gamma + beta


def _lrelu(x):
    return jnp.where(x >= 0, x, 0.2 * x)


def _tc_a_body(x_ref, pw, pb, h0w, h0b, h1w, h1b, h2w, h2b, qw, qb,
               lng, lnb, w0, aws, awd, xl0_ref, asrc_ref, adst_ref):
    x = x_ref[...]
    t = _gelu(jnp.dot(x, pw[...], preferred_element_type=jnp.float32) + pb[...])
    for (w, b) in ((h0w, h0b), (h1w, h1b), (h2w, h2b)):
        t = _gelu(jnp.dot(t, w[...], preferred_element_type=jnp.float32) + b[...]) + t
    h = jnp.dot(t, qw[...], preferred_element_type=jnp.float32) + qb[...]
    h = _layer_norm(h, lng[...], lnb[...])
    xl = jnp.dot(h, w0[...], preferred_element_type=jnp.float32)
    xl0_ref[0] = xl[:, :128]
    xl0_ref[1] = xl[:, 128:]
    asrc = jnp.dot(xl, aws[...], preferred_element_type=jnp.float32)
    adst = jnp.dot(xl, awd[...], preferred_element_type=jnp.float32)
    z = jnp.zeros((RB, AW - 2), jnp.float32)
    asrc_ref[0] = jnp.concatenate([asrc[:, 0:2], z], axis=1)
    asrc_ref[1] = jnp.concatenate([asrc[:, 2:4], z], axis=1)
    adst_ref[0] = jnp.concatenate([adst[:, 0:2], z], axis=1)
    adst_ref[1] = jnp.concatenate([adst[:, 2:4], z], axis=1)


def _tc_a(x_pad, enc, ln1, w0, aws, awd):
    weights = [enc['pre'][0], enc['pre'][1]]
    for (w, b) in enc['hidden']:
        weights += [w, b]
    weights += [enc['post'][0], enc['post'][1], ln1[0], ln1[1], w0, aws, awd]
    in_specs = [pl.BlockSpec((RB, D_IN), lambda i: (i, 0))]
    in_specs += [_full_spec(w.shape) for w in weights]
    return pl.pallas_call(
        _tc_a_body,
        grid=(GRID,),
        in_specs=in_specs,
        out_specs=[
            pl.BlockSpec((2, RB, 128), lambda i: (0, i, 0)),
            pl.BlockSpec((2, RB, AW), lambda i: (0, i, 0)),
            pl.BlockSpec((2, RB, AW), lambda i: (0, i, 0)),
        ],
        out_shape=[
            jax.ShapeDtypeStruct((2, NP, 128), jnp.float32),
            jax.ShapeDtypeStruct((2, NP, AW), jnp.float32),
            jax.ShapeDtypeStruct((2, NP, AW), jnp.float32),
        ],
    )(x_pad, *weights)


def _zero_vmem(ref, nwords):
    ncol = ref.shape[-1]
    iota = lax.broadcasted_iota(jnp.int32, (16,), 0)
    zero = jnp.zeros((16,), jnp.float32)

    @pl.loop(0, nwords // 16, unroll=8)
    def _(i):
        base = i * 16
        r = jnp.full((16,), base // ncol, jnp.int32)
        cvec = (base % ncol) + iota
        plsc.store_scatter(ref, [r, cvec], zero)


def _sc_conv0_body(xl0, asrc_h, adst_h, srcp, dstp, msg_out, den_out,
                   out_sp, den_sp, sidx, didx, didxa, msgb, exb, asb, adb,
                   gsem, asem):
    c = lax.axis_index("c")
    s = lax.axis_index("s")
    iota = lax.broadcasted_iota(jnp.int32, (16,), 0)
    z16 = jnp.zeros((16,), jnp.int32)
    o16 = jnp.ones((16,), jnp.int32)
    cnp = jnp.full((16,), c * NP, jnp.int32)

    _zero_vmem(msgb, CH * 128)
    _zero_vmem(exb, CH * AW)

    @pl.loop(0, ROWS_PER_TILE // CH)
    def _(k):
        rows = pl.ds(s * ROWS_PER_TILE + k * CH, CH)
        pltpu.sync_copy(msgb, out_sp.at[rows])
        pltpu.sync_copy(exb, den_sp.at[rows])

    plsc.subcore_barrier()

    ebase = s * (EP // NSUB)
    nchunks = EP // NSUB // CH

    @pl.loop(0, nchunks)
    def _(k):
        eb = ebase + k * CH
        pltpu.sync_copy(srcp.at[pl.ds(eb, CH)], sidx)
        pltpu.sync_copy(dstp.at[pl.ds(eb, CH)], didx)
        for g in range(8):
            sl = pl.ds(g * 16, 16)
            sidx[sl] = sidx[sl] + cnp
            didxa[sl] = didx[sl] + cnp
        gd = pltpu.async_copy(xl0.at[sidx], msgb, gsem)
        ga = pltpu.async_copy(asrc_h.at[sidx], asb, asem)
        gb = pltpu.async_copy(adst_h.at[didxa], adb, asem)
        ga.wait()
        gb.wait()

        ex0s, ex1s, evecs = [], [], []
        for g in range(8):
            rvec = g * 16 + iota
            a0 = (plsc.load_gather(asb, [rvec, z16])
                  + plsc.load_gather(adb, [rvec, z16]))
            a1 = (plsc.load_gather(asb, [rvec, o16])
                  + plsc.load_gather(adb, [rvec, o16]))
            ex0 = jnp.exp(_lrelu(a0))
            ex1 = jnp.exp(_lrelu(a1))
            plsc.store_scatter(exb, [rvec, z16], ex0)
            plsc.store_scatter(exb, [rvec, o16], ex1)
            ex0s.append(ex0)
            ex1s.append(ex1)
            evecs.append(rvec)
        dd = pltpu.async_copy(exb, den_sp.at[didx], asem, add=True)
        gd.wait()

        @pl.loop(0, 64, unroll=4)
        def _(j):
            jf = jnp.full((16,), j, jnp.int32)
            for g in range(8):
                v = plsc.load_gather(msgb, [evecs[g], jf])
                plsc.store_scatter(msgb, [evecs[g], jf], v * ex0s[g])

        @pl.loop(64, 128, unroll=4)
        def _(j):
            jf = jnp.full((16,), j, jnp.int32)
            for g in range(8):
                v = plsc.load_gather(msgb, [evecs[g], jf])
                plsc.store_scatter(msgb, [evecs[g], jf], v * ex1s[g])

        dd.wait()
        pltpu.sync_copy(msgb, out_sp.at[didx], add=True)

    plsc.subcore_barrier()

    @pl.loop(0, ROWS_PER_TILE // CH)
    def _(k):
        rloc = pl.ds(s * ROWS_PER_TILE + k * CH, CH)
        rout = pl.ds(c * NP + s * ROWS_PER_TILE + k * CH, CH)
        pltpu.sync_copy(out_sp.at[rloc], msgb)
        pltpu.sync_copy(msgb, msg_out.at[rout])
        pltpu.sync_copy(den_sp.at[rloc], exb)
        pltpu.sync_copy(exb, den_out.at[rout])


_sc_conv0 = functools.partial(
    pl.kernel,
    out_type=[
        jax.ShapeDtypeStruct((2 * NP, 128), jnp.float32),
        jax.ShapeDtypeStruct((2 * NP, AW), jnp.float32),
    ],
    mesh=_mesh,
    scratch_types=[
        pltpu.VMEM_SHARED((NP, 128), jnp.float32),
        pltpu.VMEM_SHARED((NP, AW), jnp.float32),
        pltpu.VMEM((CH,), jnp.int32),
        pltpu.VMEM((CH,), jnp.int32),
        pltpu.VMEM((CH,), jnp.int32),
        pltpu.VMEM((CH, 128), jnp.float32),
        pltpu.VMEM((CH, AW), jnp.float32),
        pltpu.VMEM((CH, AW), jnp.float32),
        pltpu.VMEM((CH, AW), jnp.float32),
        pltpu.SemaphoreType.DMA,
        pltpu.SemaphoreType.DMA,
    ],
    compiler_params=_sc_params,
)(_sc_conv0_body)


def _sc_conv1_body(xl1, a1, srcp, dstp, msg_out, den_out,
                   out_sp, den_sp, sidx, didx, msgb, exb, asb, adb, gsem, asem):
    c = lax.axis_index("c")
    s = lax.axis_index("s")
    iota = lax.broadcasted_iota(jnp.int32, (16,), 0)
    z16 = jnp.zeros((16,), jnp.int32)
    o16 = jnp.ones((16,), jnp.int32)

    _zero_vmem(msgb, CH * HD)
    _zero_vmem(exb, CH * AW)

    @pl.loop(0, ROWS_PER_TILE // CH)
    def _(k):
        rows = pl.ds(s * ROWS_PER_TILE + k * CH, CH)
        pltpu.sync_copy(msgb, out_sp.at[rows])
        pltpu.sync_copy(exb, den_sp.at[rows])

    plsc.subcore_barrier()

    ebase = c * (EP // NCORE) + s * (EP // NCORE // NSUB)
    nchunks = EP // NCORE // NSUB // CH

    @pl.loop(0, nchunks)
    def _(k):
        eb = ebase + k * CH
        pltpu.sync_copy(srcp.at[pl.ds(eb, CH)], sidx)
        pltpu.sync_copy(dstp.at[pl.ds(eb, CH)], didx)
        gd = pltpu.async_copy(xl1.at[sidx], msgb, gsem)
        ga = pltpu.async_copy(a1.at[sidx], asb, asem)
        gb = pltpu.async_copy(a1.at[didx], adb, asem)
        ga.wait()
        gb.wait()

        exs, evecs = [], []
        for g in range(8):
            rvec = g * 16 + iota
            a = (plsc.load_gather(asb, [rvec, z16])
                 + plsc.load_gather(adb, [rvec, o16]))
            ex = jnp.exp(_lrelu(a))
            plsc.store_scatter(exb, [rvec, z16], ex)
            exs.append(ex)
            evecs.append(rvec)
        dd = pltpu.async_copy(exb, den_sp.at[didx], asem, add=True)
        gd.wait()

        @pl.loop(0, HD, unroll=4)
        def _(j):
            jf = jnp.full((16,), j, jnp.int32)
            for g in range(8):
                v = plsc.load_gather(msgb, [evecs[g], jf])
                plsc.store_scatter(msgb, [evecs[g], jf], v * exs[g])

        dd.wait()
        pltpu.sync_copy(msgb, out_sp.at[didx], add=True)

    plsc.subcore_barrier()

    @pl.loop(0, ROWS_PER_TILE // CH)
    def _(k):
        rloc = pl.ds(s * ROWS_PER_TILE + k * CH, CH)
        rout = pl.ds(c * NP + s * ROWS_PER_TILE + k * CH, CH)
        pltpu.sync_copy(out_sp.at[rloc], msgb)
        pltpu.sync_copy(msgb, msg_out.at[rout])
        pltpu.sync_copy(den_sp.at[rloc], exb)
        pltpu.sync_copy(exb, den_out.at[rout])


_sc_conv1 = functools.partial(
    pl.kernel,
    out_type=[
        jax.ShapeDtypeStruct((2 * NP, HD), jnp.float32),
        jax.ShapeDtypeStruct((2 * NP, AW), jnp.float32),
    ],
    mesh=_mesh,
    scratch_types=[
        pltpu.VMEM_SHARED((NP, HD), jnp.float32),
        pltpu.VMEM_SHARED((NP, AW), jnp.float32),
        pltpu.VMEM((CH,), jnp.int32),
        pltpu.VMEM((CH,), jnp.int32),
        pltpu.VMEM((CH, HD), jnp.float32),
        pltpu.VMEM((CH, AW), jnp.float32),
        pltpu.VMEM((CH, AW), jnp.float32),
        pltpu.VMEM((CH, AW), jnp.float32),
        pltpu.SemaphoreType.DMA,
        pltpu.SemaphoreType.DMA,
    ],
    compiler_params=_sc_params,
)(_sc_conv1_body)


def _tc_b_body(msg_ref, den_ref, asrc_ref, adst_ref, xl0_ref,
               b0, w1, a1s, a1d, xl1_ref, a1_ref):
    outs = []
    for c in range(2):
        exs = jnp.exp(_lrelu(asrc_ref[c][:, 0:2] + adst_ref[c][:, 0:2]))
        den = den_ref[c][:, 0:2] + exs
        e2 = jnp.broadcast_to(exs[:, :, None], (RB, 2, HD)).reshape(RB, 128)
        d2 = jnp.broadcast_to(den[:, :, None], (RB, 2, HD)).reshape(RB, 128)
        outs.append((msg_ref[c] + e2 * xl0_ref[c]) / (d2 + 1e-16))
    out0 = jnp.concatenate(outs, axis=1) + b0[...]
    h = jnp.where(out0 > 0, out0, jnp.exp(jnp.minimum(out0, 0.0)) - 1.0)
    xl1 = jnp.dot(h, w1[...], preferred_element_type=jnp.float32)
    xl1_ref[...] = xl1
    a1_ref[...] = jnp.concatenate(
        [jnp.dot(xl1, a1s[...], preferred_element_type=jnp.float32),
         jnp.dot(xl1, a1d[...], preferred_element_type=jnp.float32),
         jnp.zeros((RB, AW - 2), jnp.float32)], axis=1)


def _tc_b(msg0, den0, asrc0, adst0, xl0, b0, w1, a1s, a1d):
    weights = [b0, w1, a1s, a1d]
    in_specs = [
        pl.BlockSpec((2, RB, 128), lambda i: (0, i, 0)),
        pl.BlockSpec((2, RB, AW), lambda i: (0, i, 0)),
        pl.BlockSpec((2, RB, AW), lambda i: (0, i, 0)),
        pl.BlockSpec((2, RB, AW), lambda i: (0, i, 0)),
        pl.BlockSpec((2, RB, 128), lambda i: (0, i, 0)),
    ] + [_full_spec(w.shape) for w in weights]
    return pl.pallas_call(
        _tc_b_body,
        grid=(GRID,),
        in_specs=in_specs,
        out_specs=[
            pl.BlockSpec((RB, HD), lambda i: (i, 0)),
            pl.BlockSpec((RB, AW), lambda i: (i, 0)),
        ],
        out_shape=[
            jax.ShapeDtypeStruct((NP, HD), jnp.float32),
            jax.ShapeDtypeStruct((NP, AW), jnp.float32),
        ],
    )(msg0, den0, asrc0, adst0, xl0, *weights)


def _tc_c_body(msg_ref, den_ref, xl1_ref, a1_ref, b1, lng, lnb,
               pw, pb, h0w, h0b, h1w, h1b, h2w, h2b, qw, qb, out_ref):
    exs = jnp.exp(_lrelu(a1_ref[:, 0:1] + a1_ref[:, 1:2]))
    den = den_ref[0][:, 0:1] + den_ref[1][:, 0:1] + exs
    out1 = (msg_ref[0] + msg_ref[1] + exs * xl1_ref[...]) / (den + 1e-16)
    out1 = out1 + b1[...]
    h = _layer_norm(out1, lng[...], lnb[...])
    t = _gelu(jnp.dot(h, pw[...], preferred_element_type=jnp.float32) + pb[...])
    for (w, b) in ((h0w, h0b), (h1w, h1b), (h2w, h2b)):
        t = _gelu(jnp.dot(t, w[...], preferred_element_type=jnp.float32) + b[...]) + t
    out_ref[...] = jnp.dot(t, qw[...], preferred_element_type=jnp.float32) + qb[...]


def _tc_c(msg1, den1, xl1, a1, b1, ln2, dec):
    weights = [b1, ln2[0], ln2[1], dec['pre'][0], dec['pre'][1]]
    for (w, b) in dec['hidden']:
        weights += [w, b]
    weights += [dec['post'][0], dec['post'][1]]
    in_specs = [
        pl.BlockSpec((2, RB, HD), lambda i: (0, i, 0)),
        pl.BlockSpec((2, RB, AW), lambda i: (0, i, 0)),
        pl.BlockSpec((RB, HD), lambda i: (i, 0)),
        pl.BlockSpec((RB, AW), lambda i: (i, 0)),
    ] + [_full_spec(w.shape) for w in weights]
    return pl.pallas_call(
        _tc_c_body,
        grid=(GRID,),
        in_specs=in_specs,
        out_specs=pl.BlockSpec((RB, D_OUT), lambda i: (i, 0)),
        out_shape=jax.ShapeDtypeStruct((NP, D_OUT), jnp.float32),
    )(msg1, den1, xl1, a1, *weights)


def kernel(x, edge_index, params):
    x_pad = jnp.pad(x, ((0, NP - N), (0, 0)))
    pad = jnp.full((EP - E,), N, jnp.int32)
    srcp = jnp.concatenate([edge_index[0], pad])
    dstp = jnp.concatenate([edge_index[1], pad])

    g0 = params['gat'][0]
    att_s = jnp.zeros((HEADS, HD, HEADS), jnp.float32)
    att_s = att_s.at[jnp.arange(HEADS), :, jnp.arange(HEADS)].set(g0['att_src'][0])
    att_d = jnp.zeros((HEADS, HD, HEADS), jnp.float32)
    att_d = att_d.at[jnp.arange(HEADS), :, jnp.arange(HEADS)].set(g0['att_dst'][0])
    aws = att_s.reshape(HEADS * HD, HEADS)
    awd = att_d.reshape(HEADS * HD, HEADS)

    xl0, asrc0, adst0 = _tc_a(x_pad, params['enc'], params['ln1'],
                              g0['W'], aws, awd)
    msg0, den0 = _sc_conv0(xl0.reshape(2 * NP, 128),
                           asrc0.reshape(2 * NP, AW),
                           adst0.reshape(2 * NP, AW), srcp, dstp)
    msg0 = msg0.reshape(2, NP, 128)
    den0 = den0.reshape(2, NP, AW)

    g1 = params['gat'][1]
    a1s = g1['att_src'].reshape(HD, 1)
    a1d = g1['att_dst'].reshape(HD, 1)
    xl1, a1 = _tc_b(msg0, den0, asrc0, adst0, xl0, g0['bias'],
                    g1['W'], a1s, a1d)
    msg1, den1 = _sc_conv1(xl1, a1, srcp, dstp)
    msg1 = msg1.reshape(2, NP, HD)
    den1 = den1.reshape(2, NP, AW)

    outp = _tc_c(msg1, den1, xl1, a1, g1['bias'], params['ln2'], params['dec'])
    return outp[:N]


# speedup vs baseline: 7.9355x; 1.0544x over previous
import functools

import jax
import jax.numpy as jnp
from jax import lax
from jax.experimental import pallas as pl
from jax.experimental.pallas import tpu as pltpu
from jax.experimental.pallas import tpu_sc as plsc

N = 10000
NP = 10240
E = 320000
EP = 323584
CH = 64
AW = 16
NSUB = 16
NCORE = 2
D_IN = 128
HD = 64
HEADS = 4
D_OUT = 6
RB = 128
GRID = NP // RB
ROWS_PER_TILE = NP // NSUB

_mesh = plsc.VectorSubcoreMesh(
    core_axis_name="c", subcore_axis_name="s",
    num_cores=NCORE, num_subcores=NSUB)

_sc_params = pltpu.CompilerParams(
    needs_layout_passes=False, use_tc_tiling_on_sc=False)


def _full_spec(shape):
    nd = len(shape)
    return pl.BlockSpec(shape, lambda i, _nd=nd: (0,) * _nd)


def _gelu(x):
    return 0.5 * x * (1.0 + lax.erf(x * 0.7071067811865476))


def _layer_norm(x, gamma, beta, eps=1e-5):
    mu = jnp.mean(x, axis=-1, keepdims=True)
    var = jnp.mean((x - mu) ** 2, axis=-1, keepdims=True)
    return (x - mu) / jnp.sqrt(var + eps) * gamma + beta


def _lrelu(x):
    return jnp.where(x >= 0, x, 0.2 * x)


def _tc_a_body(x_ref, pw, pb, h0w, h0b, h1w, h1b, h2w, h2b, qw, qb,
               lng, lnb, w0, aws, awd, xl0_ref, asrc_ref, adst_ref):
    x = x_ref[...]
    t = _gelu(jnp.dot(x, pw[...], preferred_element_type=jnp.float32) + pb[...])
    for (w, b) in ((h0w, h0b), (h1w, h1b), (h2w, h2b)):
        t = _gelu(jnp.dot(t, w[...], preferred_element_type=jnp.float32) + b[...]) + t
    h = jnp.dot(t, qw[...], preferred_element_type=jnp.float32) + qb[...]
    h = _layer_norm(h, lng[...], lnb[...])
    xl = jnp.dot(h, w0[...], preferred_element_type=jnp.float32)
    xl0_ref[0] = xl[:, :128]
    xl0_ref[1] = xl[:, 128:]
    asrc = jnp.dot(xl, aws[...], preferred_element_type=jnp.float32)
    adst = jnp.dot(xl, awd[...], preferred_element_type=jnp.float32)
    z = jnp.zeros((RB, AW - 2), jnp.float32)
    asrc_ref[0] = jnp.concatenate([asrc[:, 0:2], z], axis=1)
    asrc_ref[1] = jnp.concatenate([asrc[:, 2:4], z], axis=1)
    adst_ref[0] = jnp.concatenate([adst[:, 0:2], z], axis=1)
    adst_ref[1] = jnp.concatenate([adst[:, 2:4], z], axis=1)


def _tc_a(x_pad, enc, ln1, w0, aws, awd):
    weights = [enc['pre'][0], enc['pre'][1]]
    for (w, b) in enc['hidden']:
        weights += [w, b]
    weights += [enc['post'][0], enc['post'][1], ln1[0], ln1[1], w0, aws, awd]
    in_specs = [pl.BlockSpec((RB, D_IN), lambda i: (i, 0))]
    in_specs += [_full_spec(w.shape) for w in weights]
    return pl.pallas_call(
        _tc_a_body,
        grid=(GRID,),
        in_specs=in_specs,
        out_specs=[
            pl.BlockSpec((2, RB, 128), lambda i: (0, i, 0)),
            pl.BlockSpec((2, RB, AW), lambda i: (0, i, 0)),
            pl.BlockSpec((2, RB, AW), lambda i: (0, i, 0)),
        ],
        out_shape=[
            jax.ShapeDtypeStruct((2, NP, 128), jnp.float32),
            jax.ShapeDtypeStruct((2, NP, AW), jnp.float32),
            jax.ShapeDtypeStruct((2, NP, AW), jnp.float32),
        ],
    )(x_pad, *weights)


def _zero_vmem(ref, nwords):
    ncol = ref.shape[-1]
    iota = lax.broadcasted_iota(jnp.int32, (16,), 0)
    zero = jnp.zeros((16,), jnp.float32)

    @pl.loop(0, nwords // 16, unroll=8)
    def _(i):
        base = i * 16
        r = jnp.full((16,), base // ncol, jnp.int32)
        cvec = (base % ncol) + iota
        plsc.store_scatter(ref, [r, cvec], zero)


def _sc_conv0_body(xl0, asrc_h, adst_h, srcp, dstp, msg_out, den_out,
                   out_sp, den_sp, sidx, didx, didxa, msgb, exb, asb, adb,
                   gsem0, gsem1, dsem):
    c = lax.axis_index("c")
    s = lax.axis_index("s")
    iota = lax.broadcasted_iota(jnp.int32, (16,), 0)
    z16 = jnp.zeros((16,), jnp.int32)
    o16 = jnp.ones((16,), jnp.int32)
    cnp = jnp.full((16,), c * NP, jnp.int32)
    gsems = (gsem0, gsem1)
    ngroups = CH // 16

    _zero_vmem(msgb.at[0], CH * 128)
    _zero_vmem(exb, CH * AW)

    @pl.loop(0, ROWS_PER_TILE // CH)
    def _(k):
        rows = pl.ds(s * ROWS_PER_TILE + k * CH, CH)
        pltpu.sync_copy(msgb.at[0], out_sp.at[rows])
        pltpu.sync_copy(exb, den_sp.at[rows])

    plsc.subcore_barrier()

    ebase = s * (EP // NSUB)
    nchunks = EP // NSUB // CH

    def load_and_launch(chunk, b):
        eb = ebase + chunk * CH
        pltpu.sync_copy(srcp.at[pl.ds(eb, CH)], sidx.at[b])
        pltpu.sync_copy(dstp.at[pl.ds(eb, CH)], didx.at[b])
        for g in range(ngroups):
            sl = pl.ds(g * 16, 16)
            sidx[b, sl] = sidx[b, sl] + cnp
            didxa[b, sl] = didx[b, sl] + cnp
        pltpu.async_copy(xl0.at[sidx.at[b]], msgb.at[b], gsems[b])
        pltpu.async_copy(asrc_h.at[sidx.at[b]], asb, gsems[b])
        pltpu.async_copy(adst_h.at[didxa.at[b]], adb, gsems[b])

    def drain_gathers(b):
        pltpu.make_async_copy(xl0.at[sidx.at[b]], msgb.at[b], gsems[b]).wait()
        pltpu.make_async_copy(asrc_h.at[sidx.at[b]], asb, gsems[b]).wait()
        pltpu.make_async_copy(adst_h.at[didxa.at[b]], adb, gsems[b]).wait()

    def process_half(b, prefetch_chunk, guard):
        drain_gathers(b)
        ex0s, ex1s, evecs = [], [], []
        for g in range(ngroups):
            rvec = g * 16 + iota
            a0 = (plsc.load_gather(asb, [rvec, z16])
                  + plsc.load_gather(adb, [rvec, z16]))
            a1 = (plsc.load_gather(asb, [rvec, o16])
                  + plsc.load_gather(adb, [rvec, o16]))
            ex0 = jnp.exp(_lrelu(a0))
            ex1 = jnp.exp(_lrelu(a1))
            plsc.store_scatter(exb, [rvec, z16], ex0)
            plsc.store_scatter(exb, [rvec, o16], ex1)
            ex0s.append(ex0)
            ex1s.append(ex1)
            evecs.append(rvec)
        dd = pltpu.async_copy(exb, den_sp.at[didx.at[b]], dsem, add=True)
        if guard is None:
            load_and_launch(prefetch_chunk, 1 - b)
        else:
            @pl.when(guard)
            def _():
                load_and_launch(prefetch_chunk, 1 - b)

        @pl.loop(0, 64, unroll=4)
        def _(j):
            jf = jnp.full((16,), j, jnp.int32)
            for g in range(ngroups):
                v = plsc.load_gather(msgb.at[b], [evecs[g], jf])
                plsc.store_scatter(msgb.at[b], [evecs[g], jf], v * ex0s[g])

        @pl.loop(64, 128, unroll=4)
        def _(j):
            jf = jnp.full((16,), j, jnp.int32)
            for g in range(ngroups):
                v = plsc.load_gather(msgb.at[b], [evecs[g], jf])
                plsc.store_scatter(msgb.at[b], [evecs[g], jf], v * ex1s[g])

        dd.wait()
        pltpu.sync_copy(msgb.at[b], out_sp.at[didx.at[b]], add=True)

    load_and_launch(0, 0)

    @pl.loop(0, nchunks // 2)
    def _(i):
        process_half(0, 2 * i + 1, None)
        process_half(1, 2 * i + 2, 2 * i + 2 < nchunks)

    plsc.subcore_barrier()

    @pl.loop(0, ROWS_PER_TILE // CH)
    def _(k):
        rloc = pl.ds(s * ROWS_PER_TILE + k * CH, CH)
        rout = pl.ds(c * NP + s * ROWS_PER_TILE + k * CH, CH)
        pltpu.sync_copy(out_sp.at[rloc], msgb.at[0])
        pltpu.sync_copy(msgb.at[0], msg_out.at[rout])
        pltpu.sync_copy(den_sp.at[rloc], exb)
        pltpu.sync_copy(exb, den_out.at[rout])


_sc_conv0 = functools.partial(
    pl.kernel,
    out_type=[
        jax.ShapeDtypeStruct((2 * NP, 128), jnp.float32),
        jax.ShapeDtypeStruct((2 * NP, AW), jnp.float32),
    ],
    mesh=_mesh,
    scratch_types=[
        pltpu.VMEM_SHARED((NP, 128), jnp.float32),
        pltpu.VMEM_SHARED((NP, AW), jnp.float32),
        pltpu.VMEM((2, CH), jnp.int32),
        pltpu.VMEM((2, CH), jnp.int32),
        pltpu.VMEM((2, CH), jnp.int32),
        pltpu.VMEM((2, CH, 128), jnp.float32),
        pltpu.VMEM((CH, AW), jnp.float32),
        pltpu.VMEM((CH, AW), jnp.float32),
        pltpu.VMEM((CH, AW), jnp.float32),
        pltpu.SemaphoreType.DMA,
        pltpu.SemaphoreType.DMA,
        pltpu.SemaphoreType.DMA,
    ],
    compiler_params=_sc_params,
)(_sc_conv0_body)


def _sc_conv1_body(xl1, a1, srcp, dstp, msg_out, den_out,
                   out_sp, den_sp, sidx, didx, msgb, exb, asb, adb,
                   gsem0, gsem1, dsem):
    c = lax.axis_index("c")
    s = lax.axis_index("s")
    iota = lax.broadcasted_iota(jnp.int32, (16,), 0)
    z16 = jnp.zeros((16,), jnp.int32)
    o16 = jnp.ones((16,), jnp.int32)
    gsems = (gsem0, gsem1)
    ngroups = CH // 16

    _zero_vmem(msgb.at[0], CH * HD)
    _zero_vmem(exb, CH * AW)

    @pl.loop(0, ROWS_PER_TILE // CH)
    def _(k):
        rows = pl.ds(s * ROWS_PER_TILE + k * CH, CH)
        pltpu.sync_copy(msgb.at[0], out_sp.at[rows])
        pltpu.sync_copy(exb, den_sp.at[rows])

    plsc.subcore_barrier()

    ebase = c * (EP // NCORE) + s * (EP // NCORE // NSUB)
    nchunks = EP // NCORE // NSUB // CH

    def load_and_launch(chunk, b):
        eb = ebase + chunk * CH
        pltpu.sync_copy(srcp.at[pl.ds(eb, CH)], sidx.at[b])
        pltpu.sync_copy(dstp.at[pl.ds(eb, CH)], didx.at[b])
        pltpu.async_copy(xl1.at[sidx.at[b]], msgb.at[b], gsems[b])
        pltpu.async_copy(a1.at[sidx.at[b]], asb, gsems[b])
        pltpu.async_copy(a1.at[didx.at[b]], adb, gsems[b])

    def drain_gathers(b):
        pltpu.make_async_copy(xl1.at[sidx.at[b]], msgb.at[b], gsems[b]).wait()
        pltpu.make_async_copy(a1.at[sidx.at[b]], asb, gsems[b]).wait()
        pltpu.make_async_copy(a1.at[didx.at[b]], adb, gsems[b]).wait()

    def process_half(b, prefetch_chunk, guard):
        drain_gathers(b)
        exs, evecs = [], []
        for g in range(ngroups):
            rvec = g * 16 + iota
            a = (plsc.load_gather(asb, [rvec, z16])
                 + plsc.load_gather(adb, [rvec, o16]))
            ex = jnp.exp(_lrelu(a))
            plsc.store_scatter(exb, [rvec, z16], ex)
            exs.append(ex)
            evecs.append(rvec)
        dd = pltpu.async_copy(exb, den_sp.at[didx.at[b]], dsem, add=True)
        if guard is None:
            load_and_launch(prefetch_chunk, 1 - b)
        else:
            @pl.when(guard)
            def _():
                load_and_launch(prefetch_chunk, 1 - b)

        @pl.loop(0, HD, unroll=4)
        def _(j):
            jf = jnp.full((16,), j, jnp.int32)
            for g in range(ngroups):
                v = plsc.load_gather(msgb.at[b], [evecs[g], jf])
                plsc.store_scatter(msgb.at[b], [evecs[g], jf], v * exs[g])

        dd.wait()
        pltpu.sync_copy(msgb.at[b], out_sp.at[didx.at[b]], add=True)

    load_and_launch(0, 0)

    @pl.loop(0, nchunks // 2)
    def _(i):
        process_half(0, 2 * i + 1, None)
        process_half(1, 2 * i + 2, 2 * i + 2 < nchunks)

    plsc.subcore_barrier()

    @pl.loop(0, ROWS_PER_TILE // CH)
    def _(k):
        rloc = pl.ds(s * ROWS_PER_TILE + k * CH, CH)
        rout = pl.ds(c * NP + s * ROWS_PER_TILE + k * CH, CH)
        pltpu.sync_copy(out_sp.at[rloc], msgb.at[0])
        pltpu.sync_copy(msgb.at[0], msg_out.at[rout])
        pltpu.sync_copy(den_sp.at[rloc], exb)
        pltpu.sync_copy(exb, den_out.at[rout])


_sc_conv1 = functools.partial(
    pl.kernel,
    out_type=[
        jax.ShapeDtypeStruct((2 * NP, HD), jnp.float32),
        jax.ShapeDtypeStruct((2 * NP, AW), jnp.float32),
    ],
    mesh=_mesh,
    scratch_types=[
        pltpu.VMEM_SHARED((NP, HD), jnp.float32),
        pltpu.VMEM_SHARED((NP, AW), jnp.float32),
        pltpu.VMEM((2, CH), jnp.int32),
        pltpu.VMEM((2, CH), jnp.int32),
        pltpu.VMEM((2, CH, HD), jnp.float32),
        pltpu.VMEM((CH, AW), jnp.float32),
        pltpu.VMEM((CH, AW), jnp.float32),
        pltpu.VMEM((CH, AW), jnp.float32),
        pltpu.SemaphoreType.DMA,
        pltpu.SemaphoreType.DMA,
        pltpu.SemaphoreType.DMA,
    ],
    compiler_params=_sc_params,
)(_sc_conv1_body)


def _tc_b_body(msg_ref, den_ref, asrc_ref, adst_ref, xl0_ref,
               b0, w1, a1s, a1d, xl1_ref, a1_ref):
    outs = []
    for c in range(2):
        exs = jnp.exp(_lrelu(asrc_ref[c][:, 0:2] + adst_ref[c][:, 0:2]))
        den = den_ref[c][:, 0:2] + exs
        e2 = jnp.broadcast_to(exs[:, :, None], (RB, 2, HD)).reshape(RB, 128)
        d2 = jnp.broadcast_to(den[:, :, None], (RB, 2, HD)).reshape(RB, 128)
        outs.append((msg_ref[c] + e2 * xl0_ref[c]) / (d2 + 1e-16))
    out0 = jnp.concatenate(outs, axis=1) + b0[...]
    h = jnp.where(out0 > 0, out0, jnp.exp(jnp.minimum(out0, 0.0)) - 1.0)
    xl1 = jnp.dot(h, w1[...], preferred_element_type=jnp.float32)
    xl1_ref[...] = xl1
    a1_ref[...] = jnp.concatenate(
        [jnp.dot(xl1, a1s[...], preferred_element_type=jnp.float32),
         jnp.dot(xl1, a1d[...], preferred_element_type=jnp.float32),
         jnp.zeros((RB, AW - 2), jnp.float32)], axis=1)


def _tc_b(msg0, den0, asrc0, adst0, xl0, b0, w1, a1s, a1d):
    weights = [b0, w1, a1s, a1d]
    in_specs = [
        pl.BlockSpec((2, RB, 128), lambda i: (0, i, 0)),
        pl.BlockSpec((2, RB, AW), lambda i: (0, i, 0)),
        pl.BlockSpec((2, RB, AW), lambda i: (0, i, 0)),
        pl.BlockSpec((2, RB, AW), lambda i: (0, i, 0)),
        pl.BlockSpec((2, RB, 128), lambda i: (0, i, 0)),
    ] + [_full_spec(w.shape) for w in weights]
    return pl.pallas_call(
        _tc_b_body,
        grid=(GRID,),
        in_specs=in_specs,
        out_specs=[
            pl.BlockSpec((RB, HD), lambda i: (i, 0)),
            pl.BlockSpec((RB, AW), lambda i: (i, 0)),
        ],
        out_shape=[
            jax.ShapeDtypeStruct((NP, HD), jnp.float32),
            jax.ShapeDtypeStruct((NP, AW), jnp.float32),
        ],
    )(msg0, den0, asrc0, adst0, xl0, *weights)


def _tc_c_body(msg_ref, den_ref, xl1_ref, a1_ref, b1, lng, lnb,
               pw, pb, h0w, h0b, h1w, h1b, h2w, h2b, qw, qb, out_ref):
    exs = jnp.exp(_lrelu(a1_ref[:, 0:1] + a1_ref[:, 1:2]))
    den = den_ref[0][:, 0:1] + den_ref[1][:, 0:1] + exs
    out1 = (msg_ref[0] + msg_ref[1] + exs * xl1_ref[...]) / (den + 1e-16)
    out1 = out1 + b1[...]
    h = _layer_norm(out1, lng[...], lnb[...])
    t = _gelu(jnp.dot(h, pw[...], preferred_element_type=jnp.float32) + pb[...])
    for (w, b) in ((h0w, h0b), (h1w, h1b), (h2w, h2b)):
        t = _gelu(jnp.dot(t, w[...], preferred_element_type=jnp.float32) + b[...]) + t
    out_ref[...] = jnp.dot(t, qw[...], preferred_element_type=jnp.float32) + qb[...]


def _tc_c(msg1, den1, xl1, a1, b1, ln2, dec):
    weights = [b1, ln2[0], ln2[1], dec['pre'][0], dec['pre'][1]]
    for (w, b) in dec['hidden']:
        weights += [w, b]
    weights += [dec['post'][0], dec['post'][1]]
    in_specs = [
        pl.BlockSpec((2, RB, HD), lambda i: (0, i, 0)),
        pl.BlockSpec((2, RB, AW), lambda i: (0, i, 0)),
        pl.BlockSpec((RB, HD), lambda i: (i, 0)),
        pl.BlockSpec((RB, AW), lambda i: (i, 0)),
    ] + [_full_spec(w.shape) for w in weights]
    return pl.pallas_call(
        _tc_c_body,
        grid=(GRID,),
        in_specs=in_specs,
        out_specs=pl.BlockSpec((RB, D_OUT), lambda i: (i, 0)),
        out_shape=jax.ShapeDtypeStruct((NP, D_OUT), jnp.float32),
    )(msg1, den1, xl1, a1, *weights)


def kernel(x, edge_index, params):
    x_pad = jnp.pad(x, ((0, NP - N), (0, 0)))
    pad = jnp.full((EP - E,), N, jnp.int32)
    srcp = jnp.concatenate([edge_index[0], pad])
    dstp = jnp.concatenate([edge_index[1], pad])

    g0 = params['gat'][0]
    att_s = jnp.zeros((HEADS, HD, HEADS), jnp.float32)
    att_s = att_s.at[jnp.arange(HEADS), :, jnp.arange(HEADS)].set(g0['att_src'][0])
    att_d = jnp.zeros((HEADS, HD, HEADS), jnp.float32)
    att_d = att_d.at[jnp.arange(HEADS), :, jnp.arange(HEADS)].set(g0['att_dst'][0])
    aws = att_s.reshape(HEADS * HD, HEADS)
    awd = att_d.reshape(HEADS * HD, HEADS)

    xl0, asrc0, adst0 = _tc_a(x_pad, params['enc'], params['ln1'],
                              g0['W'], aws, awd)
    msg0, den0 = _sc_conv0(xl0.reshape(2 * NP, 128),
                           asrc0.reshape(2 * NP, AW),
                           adst0.reshape(2 * NP, AW), srcp, dstp)
    msg0 = msg0.reshape(2, NP, 128)
    den0 = den0.reshape(2, NP, AW)

    g1 = params['gat'][1]
    a1s = g1['att_src'].reshape(HD, 1)
    a1d = g1['att_dst'].reshape(HD, 1)
    xl1, a1 = _tc_b(msg0, den0, asrc0, adst0, xl0, g0['bias'],
                    g1['W'], a1s, a1d)
    msg1, den1 = _sc_conv1(xl1, a1, srcp, dstp)
    msg1 = msg1.reshape(2, NP, HD)
    den1 = den1.reshape(2, NP, AW)

    outp = _tc_c(msg1, den1, xl1, a1, g1['bias'], params['ln2'], params['dec'])
    return outp[:N]
